# Initial kernel scaffold; baseline (speedup 1.0000x reference)
import jax
import jax.numpy as jnp
from jax.experimental import pallas as pl


def kernel(x, edge_index, W1, b1, W2, b2, W_fc, b_fc):
    raise NotImplementedError("write your pallas kernel here")


# speedup vs baseline: 10.8173x; 10.8173x over previous
import functools

import jax
import jax.numpy as jnp
from jax import lax
from jax.experimental import pallas as pl
from jax.experimental.pallas import tpu as pltpu
from jax.experimental.pallas import tpu_sc as plsc

_SC_INFO = plsc.get_sparse_core_info()
NC = _SC_INFO.num_cores
NS = _SC_INFO.num_subcores
NW = NC * NS
EB = 128
HW = 16


def _worker_id():
    return lax.axis_index("s") * NC + lax.axis_index("c")


def _sc_hist_body(chunks, rows_per_tile, dst_hbm, out_hbm, buf_v, idx_v, acc_sh):
    cid = lax.axis_index("c")
    sid = lax.axis_index("s")
    wid = _worker_id()
    n_blocks = rows_per_tile // EB

    def _fill(r, _):
        buf_v[r, :] = jnp.zeros((HW,), jnp.float32)
        return _
    lax.fori_loop(0, EB, _fill, None)
    for b in range(n_blocks):
        pltpu.sync_copy(buf_v, acc_sh.at[pl.ds(sid * rows_per_tile + b * EB, EB)])
    plsc.subcore_barrier()

    def _ones(r, _):
        buf_v[r, :] = jnp.ones((HW,), jnp.float32)
        return _
    lax.fori_loop(0, EB, _ones, None)

    def _step(j, _):
        pltpu.sync_copy(dst_hbm.at[wid, j], idx_v)
        pltpu.sync_copy(buf_v, acc_sh.at[idx_v], add=True)
        return _
    lax.fori_loop(0, chunks, _step, None)
    plsc.subcore_barrier()

    pltpu.sync_copy(acc_sh.at[pl.ds(sid * rows_per_tile, rows_per_tile)],
                    out_hbm.at[cid, pl.ds(sid * rows_per_tile, rows_per_tile)])


def _sc_scatter_body(chunks, rows_per_tile, d, h_hbm, src_hbm, dst_hbm, out_hbm,
                     rows_v, sidx_v, didx_v, acc_sh, sem):
    cid = lax.axis_index("c")
    sid = lax.axis_index("s")
    wid = _worker_id()
    n_blocks = rows_per_tile // EB

    def _zero(r, _):
        for l in range(d // 16):
            rows_v[r, pl.ds(l * 16, 16)] = jnp.zeros((16,), jnp.float32)
        return _
    lax.fori_loop(0, EB, _zero, None)
    for b in range(n_blocks):
        pltpu.sync_copy(rows_v, acc_sh.at[pl.ds(sid * rows_per_tile + b * EB, EB)])
    plsc.subcore_barrier()

    def _step(j, _):
        pltpu.sync_copy(src_hbm.at[wid, j], sidx_v)
        pltpu.sync_copy(dst_hbm.at[wid, j], didx_v)
        pltpu.async_copy(h_hbm.at[sidx_v], rows_v, sem).wait()
        pltpu.sync_copy(rows_v, acc_sh.at[didx_v], add=True)
        return _
    lax.fori_loop(0, chunks, _step, None)
    plsc.subcore_barrier()

    pltpu.sync_copy(acc_sh.at[pl.ds(sid * rows_per_tile, rows_per_tile)],
                    out_hbm.at[cid, pl.ds(sid * rows_per_tile, rows_per_tile)])


def _tc1_body(x_ref, w_ref, degp_ref, out_ref):
    deg = degp_ref[0, :, 0:1] + degp_ref[1, :, 0:1] + 1.0
    dinv = lax.rsqrt(deg)
    h = jnp.dot(x_ref[...], w_ref[...], preferred_element_type=jnp.float32)
    out_ref[...] = h * dinv


def _tc2_body(aggp_ref, hp_ref, degp_ref, b_ref, w_ref, out_ref):
    deg = degp_ref[0, :, 0:1] + degp_ref[1, :, 0:1] + 1.0
    dinv = lax.rsqrt(deg)
    z = dinv * (aggp_ref[0] + aggp_ref[1] + hp_ref[...]) + b_ref[...]
    z = jnp.maximum(z, 0.0)
    out_ref[...] = jnp.dot(z, w_ref[...], preferred_element_type=jnp.float32) * dinv


def _tc3_body(n_nodes, aggp_ref, hp_ref, degp_ref, b_ref, wfc_ref, bfc_ref,
              out_ref, acc_ref):
    i = pl.program_id(0)
    ng = pl.num_programs(0)

    @pl.when(i == 0)
    def _():
        acc_ref[...] = jnp.zeros_like(acc_ref)

    deg = degp_ref[0, :, 0:1] + degp_ref[1, :, 0:1] + 1.0
    dinv = lax.rsqrt(deg)
    z = dinv * (aggp_ref[0] + aggp_ref[1] + hp_ref[...]) + b_ref[...]
    z = jnp.maximum(z, 0.0)
    acc_ref[...] += jnp.sum(z, axis=0, keepdims=True)

    @pl.when(i == ng - 1)
    def _():
        colmean = acc_ref[...] * (1.0 / n_nodes)
        y = jnp.dot(colmean, wfc_ref[...], preferred_element_type=jnp.float32)
        out_ref[...] = jax.nn.sigmoid(y + bfc_ref[...])


def _ceil_to(v, m):
    return (v + m - 1) // m * m


@jax.jit
def kernel(x, edge_index, W1, b1, W2, b2, W_fc, b_fc):
    n, d_in = x.shape
    d_hid = W1.shape[1]
    d_out = W2.shape[1]
    e = edge_index.shape[1]

    chunks = _ceil_to(e, NW * EB) // (NW * EB)
    e_pad = NW * chunks * EB
    ei = edge_index.astype(jnp.int32)
    src = jnp.concatenate([ei[0], jnp.zeros((e_pad - e,), jnp.int32)])
    dst = jnp.concatenate([ei[1], jnp.full((e_pad - e,), n, jnp.int32)])
    src = src.reshape(NW, chunks, EB)
    dst = dst.reshape(NW, chunks, EB)

    rows_per_tile = _ceil_to(n + 1, NS * EB) // NS
    acc_rows = NS * rows_per_tile

    mesh = plsc.VectorSubcoreMesh(core_axis_name="c", subcore_axis_name="s")
    hist = pl.kernel(
        functools.partial(_sc_hist_body, chunks, rows_per_tile),
        out_type=jax.ShapeDtypeStruct((NC, acc_rows, HW), jnp.float32),
        mesh=mesh,
        scratch_types=[
            pltpu.VMEM((EB, HW), jnp.float32),
            pltpu.VMEM((EB,), jnp.int32),
            pltpu.VMEM_SHARED((acc_rows, HW), jnp.float32),
        ],
    )
    degp = hist(dst)

    def make_scatter(d):
        return pl.kernel(
            functools.partial(_sc_scatter_body, chunks, rows_per_tile, d),
            out_type=jax.ShapeDtypeStruct((NC, acc_rows, d), jnp.float32),
            mesh=mesh,
            scratch_types=[
                pltpu.VMEM((EB, d), jnp.float32),
                pltpu.VMEM((EB,), jnp.int32),
                pltpu.VMEM((EB,), jnp.int32),
                pltpu.VMEM_SHARED((acc_rows, d), jnp.float32),
                pltpu.SemaphoreType.DMA,
            ],
        )

    bn = 1000
    grid = (n // bn,)
    h1p = pl.pallas_call(
        _tc1_body,
        grid=grid,
        in_specs=[
            pl.BlockSpec((bn, d_in), lambda i: (i, 0)),
            pl.BlockSpec((d_in, d_hid), lambda i: (0, 0)),
            pl.BlockSpec((NC, bn, HW), lambda i: (0, i, 0)),
        ],
        out_specs=pl.BlockSpec((bn, d_hid), lambda i: (i, 0)),
        out_shape=jax.ShapeDtypeStruct((n, d_hid), jnp.float32),
    )(x, W1, degp)

    agg1 = make_scatter(d_hid)(h1p, src, dst)

    h2p = pl.pallas_call(
        _tc2_body,
        grid=grid,
        in_specs=[
            pl.BlockSpec((NC, bn, d_hid), lambda i: (0, i, 0)),
            pl.BlockSpec((bn, d_hid), lambda i: (i, 0)),
            pl.BlockSpec((NC, bn, HW), lambda i: (0, i, 0)),
            pl.BlockSpec((1, d_hid), lambda i: (0, 0)),
            pl.BlockSpec((d_hid, d_out), lambda i: (0, 0)),
        ],
        out_specs=pl.BlockSpec((bn, d_out), lambda i: (i, 0)),
        out_shape=jax.ShapeDtypeStruct((n, d_out), jnp.float32),
    )(agg1, h1p, degp, b1.reshape(1, d_hid), W2)

    agg2 = make_scatter(d_out)(h2p, src, dst)

    out = pl.pallas_call(
        functools.partial(_tc3_body, float(n)),
        grid=grid,
        in_specs=[
            pl.BlockSpec((NC, bn, d_out), lambda i: (0, i, 0)),
            pl.BlockSpec((bn, d_out), lambda i: (i, 0)),
            pl.BlockSpec((NC, bn, HW), lambda i: (0, i, 0)),
            pl.BlockSpec((1, d_out), lambda i: (0, 0)),
            pl.BlockSpec((d_out, 1), lambda i: (0, 0)),
            pl.BlockSpec((1, 1), lambda i: (0, 0)),
        ],
        out_specs=pl.BlockSpec((1, 1), lambda i: (0, 0)),
        out_shape=jax.ShapeDtypeStruct((1, 1), jnp.float32),
        scratch_shapes=[pltpu.VMEM((1, d_out), jnp.float32)],
    )(agg2, h2p, degp, b2.reshape(1, d_out), W_fc, b_fc.reshape(1, 1))

    return out


# speedup vs baseline: 13.0662x; 1.2079x over previous
import functools

import jax
import jax.numpy as jnp
from jax import lax
from jax.experimental import pallas as pl
from jax.experimental.pallas import tpu as pltpu
from jax.experimental.pallas import tpu_sc as plsc

_SC_INFO = plsc.get_sparse_core_info()
NC = _SC_INFO.num_cores
NS = _SC_INFO.num_subcores
NW = NC * NS
EB = 128
HW = 16


def _worker_id():
    return lax.axis_index("s") * NC + lax.axis_index("c")


def _sc_hist_body(chunks, rows_per_tile, dst_hbm, out_hbm, buf_v, idx_v, acc_sh):
    cid = lax.axis_index("c")
    sid = lax.axis_index("s")
    wid = _worker_id()
    n_blocks = rows_per_tile // EB

    def _fill(r, _):
        buf_v[r, :] = jnp.zeros((HW,), jnp.float32)
        return _
    lax.fori_loop(0, EB, _fill, None)
    for b in range(n_blocks):
        pltpu.sync_copy(buf_v, acc_sh.at[pl.ds(sid * rows_per_tile + b * EB, EB)])
    plsc.subcore_barrier()

    def _ones(r, _):
        buf_v[r, :] = jnp.ones((HW,), jnp.float32)
        return _
    lax.fori_loop(0, EB, _ones, None)

    def _step(j, _):
        pltpu.sync_copy(dst_hbm.at[wid, j], idx_v)
        pltpu.sync_copy(buf_v, acc_sh.at[idx_v], add=True)
        return _
    lax.fori_loop(0, chunks, _step, None)
    plsc.subcore_barrier()

    pltpu.sync_copy(acc_sh.at[pl.ds(sid * rows_per_tile, rows_per_tile)],
                    out_hbm.at[cid, pl.ds(sid * rows_per_tile, rows_per_tile)])


def _sc_scatter_body(chunks, rows_per_tile, d, h_hbm, src_hbm, dst_hbm, out_hbm,
                     rows0, rows1, si0, si1, di0, di1, acc_sh,
                     sem0, sem1, semi0, semi1, semd0, semd1):
    cid = lax.axis_index("c")
    sid = lax.axis_index("s")
    wid = _worker_id()
    n_blocks = rows_per_tile // EB

    def _zero(r, _):
        for l in range(d // 16):
            rows0[r, pl.ds(l * 16, 16)] = jnp.zeros((16,), jnp.float32)
        return _
    lax.fori_loop(0, EB, _zero, None)
    for b in range(n_blocks):
        pltpu.sync_copy(rows0, acc_sh.at[pl.ds(sid * rows_per_tile + b * EB, EB)])
    plsc.subcore_barrier()

    def _wait_rows(si, buf, sem):
        pltpu.make_async_copy(h_hbm.at[si], buf, sem).wait()

    def _wait_idx(buf, sem):
        pltpu.make_async_copy(src_hbm.at[0, 0], buf, sem).wait()

    pltpu.sync_copy(src_hbm.at[wid, 0], si0)
    pltpu.async_copy(h_hbm.at[si0], rows0, sem0)
    pairs = (chunks - 1) // 2

    def _pair(k, _):
        j = 2 * k
        _wait_rows(si0, rows0, sem0)
        pltpu.sync_copy(src_hbm.at[wid, j + 1], si1)
        pltpu.async_copy(h_hbm.at[si1], rows1, sem1)
        pltpu.sync_copy(dst_hbm.at[wid, j], di0)
        pltpu.sync_copy(rows0, acc_sh.at[di0], add=True)
        _wait_rows(si1, rows1, sem1)
        pltpu.sync_copy(src_hbm.at[wid, j + 2], si0)
        pltpu.async_copy(h_hbm.at[si0], rows0, sem0)
        pltpu.sync_copy(dst_hbm.at[wid, j + 1], di1)
        pltpu.sync_copy(rows1, acc_sh.at[di1], add=True)
        return _
    lax.fori_loop(0, pairs, _pair, None)
    pltpu.sync_copy(dst_hbm.at[wid, chunks - 1], di0)
    _wait_rows(si0, rows0, sem0)
    pltpu.sync_copy(rows0, acc_sh.at[di0], add=True)
    plsc.subcore_barrier()

    pltpu.sync_copy(acc_sh.at[pl.ds(sid * rows_per_tile, rows_per_tile)],
                    out_hbm.at[cid, pl.ds(sid * rows_per_tile, rows_per_tile)])


def _tc1_body(x_ref, w_ref, degp_ref, out_ref):
    deg = degp_ref[0, :, 0:1] + degp_ref[1, :, 0:1] + 1.0
    dinv = lax.rsqrt(deg)
    h = jnp.dot(x_ref[...], w_ref[...], preferred_element_type=jnp.float32)
    out_ref[...] = h * dinv


def _tc2_body(aggp_ref, hp_ref, degp_ref, b_ref, w_ref, out_ref):
    deg = degp_ref[0, :, 0:1] + degp_ref[1, :, 0:1] + 1.0
    dinv = lax.rsqrt(deg)
    z = dinv * (aggp_ref[0] + aggp_ref[1] + hp_ref[...]) + b_ref[...]
    z = jnp.maximum(z, 0.0)
    out_ref[...] = jnp.dot(z, w_ref[...], preferred_element_type=jnp.float32) * dinv


def _tc3_body(n_nodes, aggp_ref, hp_ref, degp_ref, b_ref, wfc_ref, bfc_ref,
              out_ref, acc_ref):
    i = pl.program_id(0)
    ng = pl.num_programs(0)

    @pl.when(i == 0)
    def _():
        acc_ref[...] = jnp.zeros_like(acc_ref)

    deg = degp_ref[0, :, 0:1] + degp_ref[1, :, 0:1] + 1.0
    dinv = lax.rsqrt(deg)
    z = dinv * (aggp_ref[0] + aggp_ref[1] + hp_ref[...]) + b_ref[...]
    z = jnp.maximum(z, 0.0)
    acc_ref[...] += jnp.sum(z, axis=0, keepdims=True)

    @pl.when(i == ng - 1)
    def _():
        colmean = acc_ref[...] * (1.0 / n_nodes)
        y = jnp.dot(colmean, wfc_ref[...], preferred_element_type=jnp.float32)
        out_ref[...] = jax.nn.sigmoid(y + bfc_ref[...])


def _ceil_to(v, m):
    return (v + m - 1) // m * m


@jax.jit
def kernel(x, edge_index, W1, b1, W2, b2, W_fc, b_fc):
    n, d_in = x.shape
    d_hid = W1.shape[1]
    d_out = W2.shape[1]
    e = edge_index.shape[1]

    chunks = _ceil_to(e, NW * EB) // (NW * EB)
    if chunks % 2 == 0:
        chunks += 1
    e_pad = NW * chunks * EB
    ei = edge_index.astype(jnp.int32)
    src = jnp.concatenate([ei[0], jnp.zeros((e_pad - e,), jnp.int32)])
    dst = jnp.concatenate([ei[1], jnp.full((e_pad - e,), n, jnp.int32)])
    src = jnp.concatenate(
        [src.reshape(NW, chunks, EB),
         jnp.zeros((NW, 1, EB), jnp.int32)], axis=1)
    dst = jnp.concatenate(
        [dst.reshape(NW, chunks, EB),
         jnp.full((NW, 1, EB), n, jnp.int32)], axis=1)

    rows_per_tile = _ceil_to(n + 1, NS * EB) // NS
    acc_rows = NS * rows_per_tile

    mesh = plsc.VectorSubcoreMesh(core_axis_name="c", subcore_axis_name="s")
    hist = pl.kernel(
        functools.partial(_sc_hist_body, chunks, rows_per_tile),
        out_type=jax.ShapeDtypeStruct((NC, acc_rows, HW), jnp.float32),
        mesh=mesh,
        scratch_types=[
            pltpu.VMEM((EB, HW), jnp.float32),
            pltpu.VMEM((EB,), jnp.int32),
            pltpu.VMEM_SHARED((acc_rows, HW), jnp.float32),
        ],
    )
    degp = hist(dst)

    def make_scatter(d):
        return pl.kernel(
            functools.partial(_sc_scatter_body, chunks, rows_per_tile, d),
            out_type=jax.ShapeDtypeStruct((NC, acc_rows, d), jnp.float32),
            mesh=mesh,
            scratch_types=[
                pltpu.VMEM((EB, d), jnp.float32),
                pltpu.VMEM((EB, d), jnp.float32),
                pltpu.VMEM((EB,), jnp.int32),
                pltpu.VMEM((EB,), jnp.int32),
                pltpu.VMEM((EB,), jnp.int32),
                pltpu.VMEM((EB,), jnp.int32),
                pltpu.VMEM_SHARED((acc_rows, d), jnp.float32),
                pltpu.SemaphoreType.DMA,
                pltpu.SemaphoreType.DMA,
                pltpu.SemaphoreType.DMA,
                pltpu.SemaphoreType.DMA,
                pltpu.SemaphoreType.DMA,
                pltpu.SemaphoreType.DMA,
            ],
        )

    bn = 1000
    grid = (n // bn,)
    h1p = pl.pallas_call(
        _tc1_body,
        grid=grid,
        in_specs=[
            pl.BlockSpec((bn, d_in), lambda i: (i, 0)),
            pl.BlockSpec((d_in, d_hid), lambda i: (0, 0)),
            pl.BlockSpec((NC, bn, HW), lambda i: (0, i, 0)),
        ],
        out_specs=pl.BlockSpec((bn, d_hid), lambda i: (i, 0)),
        out_shape=jax.ShapeDtypeStruct((n, d_hid), jnp.float32),
    )(x, W1, degp)

    agg1 = make_scatter(d_hid)(h1p, src, dst)

    h2p = pl.pallas_call(
        _tc2_body,
        grid=grid,
        in_specs=[
            pl.BlockSpec((NC, bn, d_hid), lambda i: (0, i, 0)),
            pl.BlockSpec((bn, d_hid), lambda i: (i, 0)),
            pl.BlockSpec((NC, bn, HW), lambda i: (0, i, 0)),
            pl.BlockSpec((1, d_hid), lambda i: (0, 0)),
            pl.BlockSpec((d_hid, d_out), lambda i: (0, 0)),
        ],
        out_specs=pl.BlockSpec((bn, d_out), lambda i: (i, 0)),
        out_shape=jax.ShapeDtypeStruct((n, d_out), jnp.float32),
    )(agg1, h1p, degp, b1.reshape(1, d_hid), W2)

    agg2 = make_scatter(d_out)(h2p, src, dst)

    out = pl.pallas_call(
        functools.partial(_tc3_body, float(n)),
        grid=grid,
        in_specs=[
            pl.BlockSpec((NC, bn, d_out), lambda i: (0, i, 0)),
            pl.BlockSpec((bn, d_out), lambda i: (i, 0)),
            pl.BlockSpec((NC, bn, HW), lambda i: (0, i, 0)),
            pl.BlockSpec((1, d_out), lambda i: (0, 0)),
            pl.BlockSpec((d_out, 1), lambda i: (0, 0)),
            pl.BlockSpec((1, 1), lambda i: (0, 0)),
        ],
        out_specs=pl.BlockSpec((1, 1), lambda i: (0, 0)),
        out_shape=jax.ShapeDtypeStruct((1, 1), jnp.float32),
        scratch_shapes=[pltpu.VMEM((1, d_out), jnp.float32)],
    )(agg2, h2p, degp, b2.reshape(1, d_out), W_fc, b_fc.reshape(1, 1))

    return out
